# baseline (speedup 1.0000x reference)
import functools
import math

import jax
import jax.numpy as jnp
import numpy as np
from jax.experimental import pallas as pl
from jax.experimental.pallas import tpu as pltpu

_VOXEL_SIZE = (0.32, 0.32, 6.0)
_SPATIAL_SIZE = (-40.96, 40.96, -40.96, 40.96, 0.0, 6.0)
_B, _N, _D_FEAT, _MLP_DIM = 2, 131072, 16, 128


def _voxel_spatial_size():
    return [
        int(math.ceil((_SPATIAL_SIZE[2 * i + 1] - _SPATIAL_SIZE[2 * i]) / _VOXEL_SIZE[i] - 1e-6))
        for i in range(3)
    ]


def _voxel_origin():
    return np.array(
        [int(math.floor(_SPATIAL_SIZE[2 * i] / _VOXEL_SIZE[i])) for i in range(3)],
        dtype=np.int32,
    )


_VSS = _voxel_spatial_size()
_VOLUME = int(np.prod(_VSS))
_ORIGIN = _voxel_origin()
_SHIFT = (_VSS[1] * _VSS[2], _VSS[2], 1)

_BLK = 1024
_NP = _B * _N


def _pointnet_body(xyz_ref, pf_ref, mask_ref, w_ref, b_ref, feat_ref, ids_ref):
    i = pl.program_id(0)
    xyz = xyz_ref[0]
    res_cols = []
    valid = mask_ref[0, 0] != 0
    pid = jnp.zeros((_BLK,), jnp.int32)
    for k in range(3):
        xk = xyz[:, k]
        pvfk = jnp.floor(xk / _VOXEL_SIZE[k])
        res_cols.append(xk - pvfk * _VOXEL_SIZE[k])
        pvk = pvfk.astype(jnp.int32) - int(_ORIGIN[k])
        valid = valid & (pvk >= 0) & (pvk < _VSS[k])
        pid = pid + pvk * _SHIFT[k]
    res = jnp.stack(res_cols, axis=-1)
    batch = (i * _BLK + jax.lax.iota(jnp.int32, _BLK)) // _N
    pid = (pid + batch * _VOLUME) * valid.astype(jnp.int32)
    ids_ref[0, 0] = pid
    cat = jnp.concatenate([pf_ref[0], res], axis=-1)
    cat = cat * valid.astype(jnp.float32)[:, None]
    out = jax.lax.dot_general(
        cat, w_ref[...], (((1,), (0,)), ((), ())),
        preferred_element_type=jnp.float32,
    )
    out = jnp.maximum(out + b_ref[0], 0.0)
    feat_ref[0] = out * valid.astype(jnp.float32)[:, None]


def _pointnet(xyz, pf, mask_i32, W, b):
    nblk = _NP // _BLK
    grid = (nblk,)
    feat, ids = pl.pallas_call(
        _pointnet_body,
        grid=grid,
        in_specs=[
            pl.BlockSpec((1, _BLK, 3), lambda i: (i, 0, 0)),
            pl.BlockSpec((1, _BLK, _D_FEAT), lambda i: (i, 0, 0)),
            pl.BlockSpec((1, 1, _BLK), lambda i: (i, 0, 0)),
            pl.BlockSpec((_D_FEAT + 3, _MLP_DIM), lambda i: (0, 0)),
            pl.BlockSpec((1, _MLP_DIM), lambda i: (0, 0)),
        ],
        out_specs=[
            pl.BlockSpec((1, _BLK, _MLP_DIM), lambda i: (i, 0, 0)),
            pl.BlockSpec((1, 1, _BLK), lambda i: (i, 0, 0)),
        ],
        out_shape=[
            jax.ShapeDtypeStruct((nblk, _BLK, _MLP_DIM), jnp.float32),
            jax.ShapeDtypeStruct((nblk, 1, _BLK), jnp.int32),
        ],
    )(
        xyz.reshape(nblk, _BLK, 3),
        pf.reshape(nblk, _BLK, _D_FEAT),
        mask_i32.reshape(nblk, 1, _BLK),
        W,
        b.reshape(1, _MLP_DIM),
    )
    return feat.reshape(_NP, _MLP_DIM), ids.reshape(_NP)


def kernel(point_xyz, point_feature, point_mask, W, b):
    feat, ids = _pointnet(
        point_xyz.astype(jnp.float32),
        point_feature,
        point_mask.astype(jnp.int32),
        W,
        b,
    )
    vox = jax.ops.segment_max(feat, ids, num_segments=_B * _VOLUME)
    vox = jnp.where(vox > -1000.0, vox, jnp.zeros_like(vox))
    return vox.reshape(_B, _VSS[0], _VSS[1], _MLP_DIM)


# speedup vs baseline: 1.0411x; 1.0411x over previous
import functools
import math

import jax
import jax.numpy as jnp
import numpy as np
from jax import lax
from jax.experimental import pallas as pl
from jax.experimental.pallas import tpu as pltpu
from jax.experimental.pallas import tpu_sc as plsc
import dataclasses

_VOXEL_SIZE = (0.32, 0.32, 6.0)
_SPATIAL_SIZE = (-40.96, 40.96, -40.96, 40.96, 0.0, 6.0)
_B, _N, _D_FEAT, _MLP_DIM = 2, 131072, 16, 128


def _voxel_spatial_size():
    return [
        int(math.ceil((_SPATIAL_SIZE[2 * i + 1] - _SPATIAL_SIZE[2 * i]) / _VOXEL_SIZE[i] - 1e-6))
        for i in range(3)
    ]


def _voxel_origin():
    return np.array(
        [int(math.floor(_SPATIAL_SIZE[2 * i] / _VOXEL_SIZE[i])) for i in range(3)],
        dtype=np.int32,
    )


_VSS = _voxel_spatial_size()
_VOLUME = int(np.prod(_VSS))
_ORIGIN = _voxel_origin()
_SHIFT = (_VSS[1] * _VSS[2], _VSS[2], 1)

_BLK = 1024
_NP = _B * _N


def _pointnet_body(xyz_ref, pf_ref, mask_ref, w_ref, b_ref, feat_ref, ids_ref):
    i = pl.program_id(0)
    xyz = xyz_ref[0]
    res_cols = []
    valid = mask_ref[0, 0] != 0
    pid = jnp.zeros((_BLK,), jnp.int32)
    for k in range(3):
        xk = xyz[:, k]
        pvfk = jnp.floor(xk / _VOXEL_SIZE[k])
        res_cols.append(xk - pvfk * _VOXEL_SIZE[k])
        pvk = pvfk.astype(jnp.int32) - int(_ORIGIN[k])
        valid = valid & (pvk >= 0) & (pvk < _VSS[k])
        pid = pid + pvk * _SHIFT[k]
    res = jnp.stack(res_cols, axis=-1)
    batch = (i * _BLK + jax.lax.iota(jnp.int32, _BLK)) // _N
    pid = (pid + batch * _VOLUME) * valid.astype(jnp.int32)
    ids_ref[0, 0] = pid
    cat = jnp.concatenate([pf_ref[0], res], axis=-1)
    cat = cat * valid.astype(jnp.float32)[:, None]
    out = jax.lax.dot_general(
        cat, w_ref[...], (((1,), (0,)), ((), ())),
        preferred_element_type=jnp.float32,
    )
    out = jnp.maximum(out + b_ref[0], 0.0)
    feat_ref[0] = out * valid.astype(jnp.float32)[:, None]


def _pointnet(xyz, pf, mask_i32, W, b):
    nblk = _NP // _BLK
    grid = (nblk,)
    feat, ids = pl.pallas_call(
        _pointnet_body,
        grid=grid,
        in_specs=[
            pl.BlockSpec((1, _BLK, 3), lambda i: (i, 0, 0)),
            pl.BlockSpec((1, _BLK, _D_FEAT), lambda i: (i, 0, 0)),
            pl.BlockSpec((1, 1, _BLK), lambda i: (i, 0, 0)),
            pl.BlockSpec((_D_FEAT + 3, _MLP_DIM), lambda i: (0, 0)),
            pl.BlockSpec((1, _MLP_DIM), lambda i: (0, 0)),
        ],
        out_specs=[
            pl.BlockSpec((1, _BLK, _MLP_DIM), lambda i: (i, 0, 0)),
            pl.BlockSpec((1, 1, _BLK), lambda i: (i, 0, 0)),
        ],
        out_shape=[
            jax.ShapeDtypeStruct((nblk, _BLK, _MLP_DIM), jnp.float32),
            jax.ShapeDtypeStruct((nblk, 1, _BLK), jnp.int32),
        ],
    )(
        xyz.reshape(nblk, _BLK, 3),
        pf.reshape(nblk, _BLK, _D_FEAT),
        mask_i32.reshape(nblk, 1, _BLK),
        W,
        b.reshape(1, _MLP_DIM),
    )
    return feat.reshape(_NP, _MLP_DIM), ids.reshape(_NP)


NW = 32
NP = 262144
PPW = NP // NW
NSEG = 131072
RSEG = NSEG // NW
SUB = 512
NSUB = RSEG // SUB
D = 128
LIDB = 8192

_MESH = plsc.VectorSubcoreMesh(core_axis_name="c", subcore_axis_name="s")
_CP = pltpu.CompilerParams()
if "needs_layout_passes" in pltpu.CompilerParams.__dataclass_fields__:
    _CP = dataclasses.replace(_CP, needs_layout_passes=False)


def _permute(x, idx):
    dnums = lax.GatherDimensionNumbers(
        offset_dims=(), collapsed_slice_dims=(0,), start_index_map=(0,))
    return lax.gather(x, idx[:, None], dnums, (1,),
                      mode=lax.GatherScatterMode.PROMISE_IN_BOUNDS)


@functools.partial(
    pl.kernel,
    out_type=[
        jax.ShapeDtypeStruct((NP,), jnp.int32),
        jax.ShapeDtypeStruct((NW * NW,), jnp.int32),
    ],
    mesh=_MESH,
    compiler_params=_CP,
    scratch_types=[
        pltpu.VMEM((PPW,), jnp.int32),
        pltpu.VMEM((PPW,), jnp.int32),
        pltpu.VMEM((NW,), jnp.int32),
        pltpu.VMEM((NW,), jnp.int32),
        pltpu.SemaphoreType.DMA,
    ],
)
def bin_kernel(ids_hbm, binned_hbm, counts_hbm, idsv, binv, cnt, off, sem):
    wid = lax.axis_index("s") * 2 + lax.axis_index("c")
    iota = lax.iota(jnp.int32, 16)
    zeros = jnp.zeros((16,), jnp.int32)
    ones = jnp.ones((16,), jnp.int32)

    pltpu.sync_copy(ids_hbm.at[pl.ds(wid * PPW, PPW)], idsv)

    cnt[pl.ds(0, 16)] = zeros
    cnt[pl.ds(16, 16)] = zeros

    @pl.loop(0, PPW, step=16)
    def _hist(i):
        v = idsv[pl.ds(i, 16)]
        r = lax.shift_right_logical(v, 12)
        plsc.addupdate_scatter(cnt, [r], ones)

    c0 = cnt[pl.ds(0, 16)]
    c1 = cnt[pl.ds(16, 16)]
    s0 = plsc.cumsum(c0)
    s1 = plsc.cumsum(c1)
    tot0 = lax.reduce_max(s0, (0,))
    off[pl.ds(0, 16)] = s0 - c0
    off[pl.ds(16, 16)] = s1 - c1 + tot0

    @pl.loop(0, PPW, step=16)
    def _place(i):
        v = idsv[pl.ds(i, 16)]
        r = lax.shift_right_logical(v, 12)
        pidx = wid * PPW + i + iota
        packed = pidx * LIDB + (v & 4095)
        skey, spacked = plsc.sort_key_val(r, packed)
        prev = _permute(skey, jnp.maximum(iota - 1, 0))
        boundary = (iota == 0) | (skey != prev)
        starts = jnp.where(boundary, iota, 0)
        rank = iota - plsc.cummax(starts)
        base = plsc.load_gather(off, [skey])
        plsc.store_scatter(binv, [base + rank], spacked)
        plsc.addupdate_scatter(off, [skey], ones)

    pltpu.sync_copy(binv, binned_hbm.at[pl.ds(wid * PPW, PPW)])
    pltpu.sync_copy(cnt, counts_hbm.at[pl.ds(wid * NW, NW)])


WL_CAP = 640
GB = 64
GRID_W = (SUB + 1) * D


@functools.partial(
    pl.kernel,
    out_type=jax.ShapeDtypeStruct((NSEG * D,), jnp.float32),
    mesh=_MESH,
    compiler_params=_CP,
    scratch_types=[
        pltpu.VMEM((NW * NW,), jnp.int32),
        pltpu.VMEM((512,), jnp.int32),
        pltpu.VMEM((WL_CAP,), jnp.int32),
        pltpu.VMEM((GB,), jnp.int32),
        pltpu.VMEM((GB,), jnp.int32),
        pltpu.VMEM((GB, D), jnp.float32),
        pltpu.VMEM((GRID_W,), jnp.float32),
        pltpu.SMEM((NW,), jnp.int32),
        pltpu.SMEM((NW,), jnp.int32),
        pltpu.SemaphoreType.DMA,
    ],
)
def scatter_max_kernel(binned_hbm, counts_hbm, feat_hbm, out_hbm,
                       cntv, chunkv, wl, gidx, lidb, rows, grid,
                       startsm, lensm, sem):
    wid = lax.axis_index("s") * 2 + lax.axis_index("c")
    iota = lax.iota(jnp.int32, 16)
    zeros = jnp.zeros((16,), jnp.int32)
    fzeros = jnp.zeros((16,), jnp.float32)

    pltpu.sync_copy(counts_hbm, cntv)

    @pl.loop(0, NW)
    def _win(s):
        r0 = cntv[pl.ds(s * NW, 16)]
        r1 = cntv[pl.ds(s * NW + 16, 16)]
        lo0 = jnp.where(iota < wid, r0, 0)
        lo1 = jnp.where(iota + 16 < wid, r1, 0)
        o = lax.reduce_sum(lo0, (0,)) + lax.reduce_sum(lo1, (0,))
        e0 = jnp.where(iota == wid, r0, 0)
        e1 = jnp.where(iota + 16 == wid, r1, 0)
        c = lax.reduce_sum(e0, (0,)) + lax.reduce_sum(e1, (0,))
        startsm[s] = o
        lensm[s] = c

    def drain(fr, nreal, sub_base):
        nreal_v = jnp.full((16,), nreal, jnp.int32)
        for j in range(GB // 16):
            v = wl[pl.ds(fr + 16 * j, 16)]
            validm = (16 * j + iota) < nreal_v
            sent = (wid * GB + iota) * LIDB + (sub_base + SUB)
            pv = jnp.where(validm, v, sent)
            gidx[pl.ds(16 * j, 16)] = lax.shift_right_logical(pv, 13)
            lidb[pl.ds(16 * j, 16)] = pv & (LIDB - 1)
        pltpu.async_copy(feat_hbm.at[gidx], rows, sem).wait()
        sb_v = jnp.full((16,), sub_base, jnp.int32)

        @pl.loop(0, GB)
        def _rmw(e):
            e16 = pl.multiple_of(e & -16, 8)
            lv = lidb[pl.ds(e16, 16)]
            lane = jnp.full((16,), e & 15, jnp.int32)
            lid_splat = _permute(lv, lane)
            addr0 = (lid_splat - sb_v) * D
            ev = jnp.full((16,), e, jnp.int32)
            for k in range(D // 16):
                a = addr0 + 16 * k + iota
                g = plsc.load_gather(grid, [a])
                rv = plsc.load_gather(rows, [ev, 16 * k + iota])
                plsc.store_scatter(grid, [a], jnp.maximum(g, rv))

    @pl.loop(0, NSUB)
    def _subpass(c):
        sub_base = c * SUB

        @pl.loop(0, SUB * D, step=256)
        def _zero(i):
            for k in range(16):
                grid[pl.ds(i + 16 * k, 16)] = fzeros

        def _per_src(s, bkv):
            o = startsm[s]
            cn = lensm[s]
            a8 = o & ~7
            end = o + cn
            nch = lax.div(end - a8 + 511, 512)

            def _chunk(k, bkv):
                base = jnp.minimum(a8 + 512 * k, PPW - 512)
                off8 = pl.multiple_of(s * PPW + base, 8)
                pltpu.sync_copy(binned_hbm.at[pl.ds(off8, 512)], chunkv)
                o_v = jnp.full((16,), o, jnp.int32)
                e_v = jnp.full((16,), end, jnp.int32)
                lo_v = jnp.full((16,), sub_base, jnp.int32)
                hi_v = jnp.full((16,), sub_base + SUB, jnp.int32)
                base_v = jnp.full((16,), base, jnp.int32)

                def _vec(j, bkv):
                    v = chunkv[pl.ds(j * 16, 16)]
                    posr = base_v + j * 16 + iota
                    lid13 = v & (LIDB - 1)
                    m = ((posr >= o_v) & (posr < e_v)
                         & (lid13 >= lo_v) & (lid13 < hi_v))
                    mi = m.astype(jnp.int32)
                    pos = bkv + plsc.cumsum(mi) - 1
                    plsc.store_scatter(wl, [pos], v, mask=m)
                    return bkv + plsc.all_reduce_population_count(m)

                bkv = lax.fori_loop(0, 32, _vec, bkv)
                bk = lax.reduce_max(bkv, (0,))
                nb = lax.shift_right_logical(bk, 6)

                def _dr(d, _):
                    drain(d * GB, GB, sub_base)
                    return 0
                lax.fori_loop(0, nb, _dr, 0)
                rem = bk - nb * GB

                @pl.when(nb > 0)
                def _compact():
                    for j in range(4):
                        wv = wl[pl.ds(nb * GB + 16 * j, 16)]
                        wl[pl.ds(16 * j, 16)] = wv
                return jnp.full((16,), rem, jnp.int32)

            return lax.fori_loop(0, nch, _chunk, bkv)

        bkv = lax.fori_loop(0, NW, _per_src, zeros)
        bk = lax.reduce_max(bkv, (0,))

        @pl.when(bk > 0)
        def _final():
            drain(0, bk, sub_base)

        out_off = pl.multiple_of((wid * RSEG + sub_base) * D, 8)
        pltpu.sync_copy(
            grid.at[pl.ds(0, SUB * D)],
            out_hbm.at[pl.ds(out_off, SUB * D)])


def segment_max_sc(feat, ids):
    binned, counts = bin_kernel(ids)
    out = scatter_max_kernel(binned, counts, feat)
    return out.reshape(NSEG, D)


def kernel(point_xyz, point_feature, point_mask, W, b):
    feat, ids = _pointnet(
        point_xyz.astype(jnp.float32),
        point_feature,
        point_mask.astype(jnp.int32),
        W,
        b,
    )
    vox = segment_max_sc(feat, ids)
    return vox.reshape(_B, _VSS[0], _VSS[1], _MLP_DIM)


# speedup vs baseline: 1.6407x; 1.5760x over previous
import functools
import math

import jax
import jax.numpy as jnp
import numpy as np
from jax import lax
from jax.experimental import pallas as pl
from jax.experimental.pallas import tpu as pltpu
from jax.experimental.pallas import tpu_sc as plsc
import dataclasses

_VOXEL_SIZE = (0.32, 0.32, 6.0)
_SPATIAL_SIZE = (-40.96, 40.96, -40.96, 40.96, 0.0, 6.0)
_B, _N, _D_FEAT, _MLP_DIM = 2, 131072, 16, 128


def _voxel_spatial_size():
    return [
        int(math.ceil((_SPATIAL_SIZE[2 * i + 1] - _SPATIAL_SIZE[2 * i]) / _VOXEL_SIZE[i] - 1e-6))
        for i in range(3)
    ]


def _voxel_origin():
    return np.array(
        [int(math.floor(_SPATIAL_SIZE[2 * i] / _VOXEL_SIZE[i])) for i in range(3)],
        dtype=np.int32,
    )


_VSS = _voxel_spatial_size()
_VOLUME = int(np.prod(_VSS))
_ORIGIN = _voxel_origin()
_SHIFT = (_VSS[1] * _VSS[2], _VSS[2], 1)

_BLK = 2048
_NP = _B * _N


def _k1_body(xb, yb, zb, mb, ids_ref, r_ref):
    i = pl.program_id(0)
    cols = (xb[0], yb[0], zb[0])
    valid = mb[0] != 0
    pid = jnp.zeros((16, 128), jnp.int32)
    for k in range(3):
        xk = cols[k]
        pvfk = jnp.floor(xk / _VOXEL_SIZE[k])
        r_ref[0, k] = xk - pvfk * _VOXEL_SIZE[k]
        pvk = pvfk.astype(jnp.int32) - int(_ORIGIN[k])
        valid = valid & (pvk >= 0) & (pvk < _VSS[k])
        pid = pid + pvk * _SHIFT[k]
    gidx = (i * _BLK + lax.broadcasted_iota(jnp.int32, (16, 128), 0) * 128
            + lax.broadcasted_iota(jnp.int32, (16, 128), 1))
    pid = (pid + (gidx // _N) * _VOLUME) * valid.astype(jnp.int32)
    ids_ref[0] = pid
    r_ref[0, 3] = 1.0 - valid.astype(jnp.float32)


def _k2_body(pf_ref, rm_ref, w16_ref, w4_ref, b_ref, feat_ref):
    acc = jax.lax.dot_general(
        pf_ref[0], w16_ref[...], (((1,), (0,)), ((), ())),
        preferred_element_type=jnp.float32,
    )
    acc = acc + jax.lax.dot_general(
        rm_ref[0], w4_ref[...], (((0,), (0,)), ((), ())),
        preferred_element_type=jnp.float32,
    )
    feat_ref[0] = jnp.maximum(acc + b_ref[0], 0.0)


def _pointnet(xyz, pf, mask_i32, W, b):
    nblk = _NP // _BLK
    xyz_t = xyz.reshape(_NP, 3).T.reshape(3, nblk, 16, 128)
    mask3 = mask_i32.reshape(nblk, 16, 128)
    ids, rm = pl.pallas_call(
        _k1_body,
        grid=(nblk,),
        in_specs=[
            pl.BlockSpec((1, 16, 128), lambda i: (i, 0, 0)) for _ in range(4)
        ],
        out_specs=[
            pl.BlockSpec((1, 16, 128), lambda i: (i, 0, 0)),
            pl.BlockSpec((1, 4, 16, 128), lambda i: (i, 0, 0, 0)),
        ],
        out_shape=[
            jax.ShapeDtypeStruct((nblk, 16, 128), jnp.int32),
            jax.ShapeDtypeStruct((nblk, 4, 16, 128), jnp.float32),
        ],
    )(xyz_t[0], xyz_t[1], xyz_t[2], mask3)
    w4 = jnp.concatenate(
        [W[_D_FEAT:], jnp.full((1, _MLP_DIM), -1e30, jnp.float32)], axis=0)
    feat = pl.pallas_call(
        _k2_body,
        grid=(nblk,),
        in_specs=[
            pl.BlockSpec((1, _BLK, _D_FEAT), lambda i: (i, 0, 0)),
            pl.BlockSpec((1, 4, _BLK), lambda i: (i, 0, 0)),
            pl.BlockSpec((_D_FEAT, _MLP_DIM), lambda i: (0, 0)),
            pl.BlockSpec((4, _MLP_DIM), lambda i: (0, 0)),
            pl.BlockSpec((1, _MLP_DIM), lambda i: (0, 0)),
        ],
        out_specs=pl.BlockSpec((1, _BLK, _MLP_DIM), lambda i: (i, 0, 0)),
        out_shape=jax.ShapeDtypeStruct((nblk, _BLK, _MLP_DIM), jnp.float32),
    )(
        pf.reshape(nblk, _BLK, _D_FEAT),
        rm.reshape(nblk, 4, _BLK),
        W[:_D_FEAT],
        w4,
        b.reshape(1, _MLP_DIM),
    )
    return feat.reshape(_NP, _MLP_DIM), ids.reshape(_NP)


NW = 32
NP = 262144
PPW = NP // NW
NSEG = 131072
RSEG = NSEG // NW
SUB = 512
NSUB = RSEG // SUB
D = 128
LIDB = 8192

_MESH = plsc.VectorSubcoreMesh(core_axis_name="c", subcore_axis_name="s")
_CP = pltpu.CompilerParams()
if "needs_layout_passes" in pltpu.CompilerParams.__dataclass_fields__:
    _CP = dataclasses.replace(_CP, needs_layout_passes=False)


def _permute(x, idx):
    dnums = lax.GatherDimensionNumbers(
        offset_dims=(), collapsed_slice_dims=(0,), start_index_map=(0,))
    return lax.gather(x, idx[:, None], dnums, (1,),
                      mode=lax.GatherScatterMode.PROMISE_IN_BOUNDS)


@functools.partial(
    pl.kernel,
    out_type=[
        jax.ShapeDtypeStruct((NP,), jnp.int32),
        jax.ShapeDtypeStruct((NW * NW,), jnp.int32),
    ],
    mesh=_MESH,
    compiler_params=_CP,
    scratch_types=[
        pltpu.VMEM((PPW,), jnp.int32),
        pltpu.VMEM((PPW,), jnp.int32),
        pltpu.VMEM((NW,), jnp.int32),
        pltpu.VMEM((NW,), jnp.int32),
        pltpu.SemaphoreType.DMA,
    ],
)
def bin_kernel(ids_hbm, binned_hbm, counts_hbm, idsv, binv, cnt, off, sem):
    wid = lax.axis_index("s") * 2 + lax.axis_index("c")
    iota = lax.iota(jnp.int32, 16)
    zeros = jnp.zeros((16,), jnp.int32)
    ones = jnp.ones((16,), jnp.int32)

    pltpu.sync_copy(ids_hbm.at[pl.ds(wid * PPW, PPW)], idsv)

    cnt[pl.ds(0, 16)] = zeros
    cnt[pl.ds(16, 16)] = zeros

    @pl.loop(0, PPW, step=16)
    def _hist(i):
        v = idsv[pl.ds(i, 16)]
        r = lax.shift_right_logical(v, 12)
        plsc.addupdate_scatter(cnt, [r], ones)

    c0 = cnt[pl.ds(0, 16)]
    c1 = cnt[pl.ds(16, 16)]
    s0 = plsc.cumsum(c0)
    s1 = plsc.cumsum(c1)
    tot0 = lax.reduce_max(s0, (0,))
    off[pl.ds(0, 16)] = s0 - c0
    off[pl.ds(16, 16)] = s1 - c1 + tot0

    @pl.loop(0, PPW, step=16)
    def _place(i):
        v = idsv[pl.ds(i, 16)]
        r = lax.shift_right_logical(v, 12)
        pidx = wid * PPW + i + iota
        packed = pidx * LIDB + (v & 4095)
        skey, spacked = plsc.sort_key_val(r, packed)
        prev = _permute(skey, jnp.maximum(iota - 1, 0))
        boundary = (iota == 0) | (skey != prev)
        starts = jnp.where(boundary, iota, 0)
        rank = iota - plsc.cummax(starts)
        base = plsc.load_gather(off, [skey])
        plsc.store_scatter(binv, [base + rank], spacked)
        plsc.addupdate_scatter(off, [skey], ones)

    pltpu.sync_copy(binv, binned_hbm.at[pl.ds(wid * PPW, PPW)])
    pltpu.sync_copy(cnt, counts_hbm.at[pl.ds(wid * NW, NW)])


WL_CAP = 640
GB = 64
GRID_W = (SUB + 1) * D


@functools.partial(
    pl.kernel,
    out_type=jax.ShapeDtypeStruct((NSEG * D,), jnp.float32),
    mesh=_MESH,
    compiler_params=_CP,
    scratch_types=[
        pltpu.VMEM((NW * NW,), jnp.int32),
        pltpu.VMEM((512,), jnp.int32),
        pltpu.VMEM((WL_CAP,), jnp.int32),
        pltpu.VMEM((GB,), jnp.int32),
        pltpu.VMEM((GB,), jnp.int32),
        pltpu.VMEM((GB, D), jnp.float32),
        pltpu.VMEM((GRID_W,), jnp.float32),
        pltpu.SMEM((NW,), jnp.int32),
        pltpu.SMEM((NW,), jnp.int32),
        pltpu.SemaphoreType.DMA,
    ],
)
def scatter_max_kernel(binned_hbm, counts_hbm, feat_hbm, out_hbm,
                       cntv, chunkv, wl, gidx, lidb, rows, grid,
                       startsm, lensm, sem):
    wid = lax.axis_index("s") * 2 + lax.axis_index("c")
    iota = lax.iota(jnp.int32, 16)
    zeros = jnp.zeros((16,), jnp.int32)
    fzeros = jnp.zeros((16,), jnp.float32)

    pltpu.sync_copy(counts_hbm, cntv)

    @pl.loop(0, NW)
    def _win(s):
        r0 = cntv[pl.ds(s * NW, 16)]
        r1 = cntv[pl.ds(s * NW + 16, 16)]
        lo0 = jnp.where(iota < wid, r0, 0)
        lo1 = jnp.where(iota + 16 < wid, r1, 0)
        o = lax.reduce_sum(lo0, (0,)) + lax.reduce_sum(lo1, (0,))
        e0 = jnp.where(iota == wid, r0, 0)
        e1 = jnp.where(iota + 16 == wid, r1, 0)
        c = lax.reduce_sum(e0, (0,)) + lax.reduce_sum(e1, (0,))
        startsm[s] = o
        lensm[s] = c

    def drain(fr, nreal, sub_base):
        nreal_v = jnp.full((16,), nreal, jnp.int32)
        for j in range(GB // 16):
            v = wl[pl.ds(fr + 16 * j, 16)]
            validm = (16 * j + iota) < nreal_v
            sent = (wid * GB + iota) * LIDB + (sub_base + SUB)
            pv = jnp.where(validm, v, sent)
            gidx[pl.ds(16 * j, 16)] = lax.shift_right_logical(pv, 13)
            lidb[pl.ds(16 * j, 16)] = pv & (LIDB - 1)
        pltpu.async_copy(feat_hbm.at[gidx], rows, sem).wait()
        sb_v = jnp.full((16,), sub_base, jnp.int32)

        @pl.loop(0, GB)
        def _rmw(e):
            e16 = pl.multiple_of(e & -16, 8)
            lv = lidb[pl.ds(e16, 16)]
            lane = jnp.full((16,), e & 15, jnp.int32)
            lid_splat = _permute(lv, lane)
            addr0 = (lid_splat - sb_v) * D
            ev = jnp.full((16,), e, jnp.int32)
            for k in range(D // 16):
                a = addr0 + 16 * k + iota
                g = plsc.load_gather(grid, [a])
                rv = plsc.load_gather(rows, [ev, 16 * k + iota])
                plsc.store_scatter(grid, [a], jnp.maximum(g, rv))

    @pl.loop(0, NSUB)
    def _subpass(c):
        sub_base = c * SUB

        @pl.loop(0, SUB * D, step=256)
        def _zero(i):
            for k in range(16):
                grid[pl.ds(i + 16 * k, 16)] = fzeros

        def _per_src(s, bkv):
            o = startsm[s]
            cn = lensm[s]
            a8 = o & ~7
            end = o + cn
            nch = lax.div(end - a8 + 511, 512)

            def _chunk(k, bkv):
                base = jnp.minimum(a8 + 512 * k, PPW - 512)
                off8 = pl.multiple_of(s * PPW + base, 8)
                pltpu.sync_copy(binned_hbm.at[pl.ds(off8, 512)], chunkv)
                o_v = jnp.full((16,), o, jnp.int32)
                e_v = jnp.full((16,), end, jnp.int32)
                lo_v = jnp.full((16,), sub_base, jnp.int32)
                hi_v = jnp.full((16,), sub_base + SUB, jnp.int32)
                base_v = jnp.full((16,), base, jnp.int32)

                def _vec(j, bkv):
                    v = chunkv[pl.ds(j * 16, 16)]
                    posr = base_v + j * 16 + iota
                    lid13 = v & (LIDB - 1)
                    m = ((posr >= o_v) & (posr < e_v)
                         & (lid13 >= lo_v) & (lid13 < hi_v))
                    mi = m.astype(jnp.int32)
                    pos = bkv + plsc.cumsum(mi) - 1
                    plsc.store_scatter(wl, [pos], v, mask=m)
                    return bkv + plsc.all_reduce_population_count(m)

                bkv = lax.fori_loop(0, 32, _vec, bkv)
                bk = lax.reduce_max(bkv, (0,))
                nb = lax.shift_right_logical(bk, 6)

                def _dr(d, _):
                    drain(d * GB, GB, sub_base)
                    return 0
                lax.fori_loop(0, nb, _dr, 0)
                rem = bk - nb * GB

                @pl.when(nb > 0)
                def _compact():
                    for j in range(4):
                        wv = wl[pl.ds(nb * GB + 16 * j, 16)]
                        wl[pl.ds(16 * j, 16)] = wv
                return jnp.full((16,), rem, jnp.int32)

            return lax.fori_loop(0, nch, _chunk, bkv)

        bkv = lax.fori_loop(0, NW, _per_src, zeros)
        bk = lax.reduce_max(bkv, (0,))

        @pl.when(bk > 0)
        def _final():
            drain(0, bk, sub_base)

        out_off = pl.multiple_of((wid * RSEG + sub_base) * D, 8)
        pltpu.sync_copy(
            grid.at[pl.ds(0, SUB * D)],
            out_hbm.at[pl.ds(out_off, SUB * D)])


def segment_max_sc(feat, ids):
    binned, counts = bin_kernel(ids)
    out = scatter_max_kernel(binned, counts, feat)
    return out.reshape(NSEG, D)


def kernel(point_xyz, point_feature, point_mask, W, b):
    feat, ids = _pointnet(
        point_xyz.astype(jnp.float32),
        point_feature,
        point_mask.astype(jnp.int32),
        W,
        b,
    )
    vox = segment_max_sc(feat, ids)
    return vox.reshape(_B, _VSS[0], _VSS[1], _MLP_DIM)


# speedup vs baseline: 1.9734x; 1.2028x over previous
import functools
import math

import jax
import jax.numpy as jnp
import numpy as np
from jax import lax
from jax.experimental import pallas as pl
from jax.experimental.pallas import tpu as pltpu
from jax.experimental.pallas import tpu_sc as plsc
import dataclasses

_VOXEL_SIZE = (0.32, 0.32, 6.0)
_SPATIAL_SIZE = (-40.96, 40.96, -40.96, 40.96, 0.0, 6.0)
_B, _N, _D_FEAT, _MLP_DIM = 2, 131072, 16, 128


def _voxel_spatial_size():
    return [
        int(math.ceil((_SPATIAL_SIZE[2 * i + 1] - _SPATIAL_SIZE[2 * i]) / _VOXEL_SIZE[i] - 1e-6))
        for i in range(3)
    ]


def _voxel_origin():
    return np.array(
        [int(math.floor(_SPATIAL_SIZE[2 * i] / _VOXEL_SIZE[i])) for i in range(3)],
        dtype=np.int32,
    )


_VSS = _voxel_spatial_size()
_VOLUME = int(np.prod(_VSS))
_ORIGIN = _voxel_origin()
_SHIFT = (_VSS[1] * _VSS[2], _VSS[2], 1)

_BLK = 2048
_NP = _B * _N


def _k1_body(xb, yb, zb, mb, ids_ref, r_ref):
    i = pl.program_id(0)
    cols = (xb[0], yb[0], zb[0])
    valid = mb[0] != 0
    pid = jnp.zeros((16, 128), jnp.int32)
    for k in range(3):
        xk = cols[k]
        pvfk = jnp.floor(xk / _VOXEL_SIZE[k])
        r_ref[0, k] = xk - pvfk * _VOXEL_SIZE[k]
        pvk = pvfk.astype(jnp.int32) - int(_ORIGIN[k])
        valid = valid & (pvk >= 0) & (pvk < _VSS[k])
        pid = pid + pvk * _SHIFT[k]
    gidx = (i * _BLK + lax.broadcasted_iota(jnp.int32, (16, 128), 0) * 128
            + lax.broadcasted_iota(jnp.int32, (16, 128), 1))
    pid = (pid + (gidx // _N) * _VOLUME) * valid.astype(jnp.int32)
    ids_ref[0] = pid
    r_ref[0, 3] = 1.0 - valid.astype(jnp.float32)


def _k2_body(pf_ref, rm_ref, w16_ref, w4_ref, b_ref, feat_ref):
    acc = jax.lax.dot_general(
        pf_ref[0], w16_ref[...], (((1,), (0,)), ((), ())),
        preferred_element_type=jnp.float32,
    )
    acc = acc + jax.lax.dot_general(
        rm_ref[0], w4_ref[...], (((0,), (0,)), ((), ())),
        preferred_element_type=jnp.float32,
    )
    feat_ref[0] = jnp.maximum(acc + b_ref[0], 0.0)


def _pointnet(xyz, pf, mask_i32, W, b):
    nblk = _NP // _BLK
    xyz_t = xyz.reshape(_NP, 3).T.reshape(3, nblk, 16, 128)
    mask3 = mask_i32.reshape(nblk, 16, 128)
    ids, rm = pl.pallas_call(
        _k1_body,
        grid=(nblk,),
        in_specs=[
            pl.BlockSpec((1, 16, 128), lambda i: (i, 0, 0)) for _ in range(4)
        ],
        out_specs=[
            pl.BlockSpec((1, 16, 128), lambda i: (i, 0, 0)),
            pl.BlockSpec((1, 4, 16, 128), lambda i: (i, 0, 0, 0)),
        ],
        out_shape=[
            jax.ShapeDtypeStruct((nblk, 16, 128), jnp.int32),
            jax.ShapeDtypeStruct((nblk, 4, 16, 128), jnp.float32),
        ],
    )(xyz_t[0], xyz_t[1], xyz_t[2], mask3)
    w4 = jnp.concatenate(
        [W[_D_FEAT:], jnp.full((1, _MLP_DIM), -1e30, jnp.float32)], axis=0)
    feat = pl.pallas_call(
        _k2_body,
        grid=(nblk,),
        in_specs=[
            pl.BlockSpec((1, _BLK, _D_FEAT), lambda i: (i, 0, 0)),
            pl.BlockSpec((1, 4, _BLK), lambda i: (i, 0, 0)),
            pl.BlockSpec((_D_FEAT, _MLP_DIM), lambda i: (0, 0)),
            pl.BlockSpec((4, _MLP_DIM), lambda i: (0, 0)),
            pl.BlockSpec((1, _MLP_DIM), lambda i: (0, 0)),
        ],
        out_specs=pl.BlockSpec((1, _BLK, _MLP_DIM), lambda i: (i, 0, 0)),
        out_shape=jax.ShapeDtypeStruct((nblk, _BLK, _MLP_DIM), jnp.float32),
    )(
        pf.reshape(nblk, _BLK, _D_FEAT),
        rm.reshape(nblk, 4, _BLK),
        W[:_D_FEAT],
        w4,
        b.reshape(1, _MLP_DIM),
    )
    return feat.reshape(_NP, _MLP_DIM), ids.reshape(_NP)


NW = 32
NP = 262144
PPW = NP // NW
NSEG = 131072
RSEG = NSEG // NW
SUB = 512
NSUB = RSEG // SUB
D = 128
LIDB = 8192

_MESH = plsc.VectorSubcoreMesh(core_axis_name="c", subcore_axis_name="s")
_CP = pltpu.CompilerParams()
if "needs_layout_passes" in pltpu.CompilerParams.__dataclass_fields__:
    _CP = dataclasses.replace(_CP, needs_layout_passes=False)


def _permute(x, idx):
    dnums = lax.GatherDimensionNumbers(
        offset_dims=(), collapsed_slice_dims=(0,), start_index_map=(0,))
    return lax.gather(x, idx[:, None], dnums, (1,),
                      mode=lax.GatherScatterMode.PROMISE_IN_BOUNDS)


@functools.partial(
    pl.kernel,
    out_type=[
        jax.ShapeDtypeStruct((NP,), jnp.int32),
        jax.ShapeDtypeStruct((NW * NW,), jnp.int32),
    ],
    mesh=_MESH,
    compiler_params=_CP,
    scratch_types=[
        pltpu.VMEM((PPW,), jnp.int32),
        pltpu.VMEM((PPW,), jnp.int32),
        pltpu.VMEM((NW,), jnp.int32),
        pltpu.VMEM((NW,), jnp.int32),
        pltpu.SemaphoreType.DMA,
    ],
)
def bin_kernel(ids_hbm, binned_hbm, counts_hbm, idsv, binv, cnt, off, sem):
    wid = lax.axis_index("s") * 2 + lax.axis_index("c")
    iota = lax.iota(jnp.int32, 16)
    zeros = jnp.zeros((16,), jnp.int32)
    ones = jnp.ones((16,), jnp.int32)

    pltpu.sync_copy(ids_hbm.at[pl.ds(wid * PPW, PPW)], idsv)

    cnt[pl.ds(0, 16)] = zeros
    cnt[pl.ds(16, 16)] = zeros

    @pl.loop(0, PPW, step=16)
    def _hist(i):
        v = idsv[pl.ds(i, 16)]
        r = lax.shift_right_logical(v, 12)
        plsc.addupdate_scatter(cnt, [r], ones)

    c0 = cnt[pl.ds(0, 16)]
    c1 = cnt[pl.ds(16, 16)]
    s0 = plsc.cumsum(c0)
    s1 = plsc.cumsum(c1)
    tot0 = lax.reduce_max(s0, (0,))
    off[pl.ds(0, 16)] = s0 - c0
    off[pl.ds(16, 16)] = s1 - c1 + tot0

    @pl.loop(0, PPW, step=16)
    def _place(i):
        v = idsv[pl.ds(i, 16)]
        r = lax.shift_right_logical(v, 12)
        pidx = wid * PPW + i + iota
        packed = pidx * LIDB + (v & 4095)
        skey, spacked = plsc.sort_key_val(r, packed)
        prev = _permute(skey, jnp.maximum(iota - 1, 0))
        boundary = (iota == 0) | (skey != prev)
        starts = jnp.where(boundary, iota, 0)
        rank = iota - plsc.cummax(starts)
        base = plsc.load_gather(off, [skey])
        plsc.store_scatter(binv, [base + rank], spacked)
        plsc.addupdate_scatter(off, [skey], ones)

    pltpu.sync_copy(binv, binned_hbm.at[pl.ds(wid * PPW, PPW)])
    pltpu.sync_copy(cnt, counts_hbm.at[pl.ds(wid * NW, NW)])


WL_CAP = 2560
GB = 64
GRID_W = (SUB + 1) * D


@functools.partial(
    pl.kernel,
    out_type=jax.ShapeDtypeStruct((NSEG * D,), jnp.float32),
    mesh=_MESH,
    compiler_params=_CP,
    scratch_types=[
        pltpu.VMEM((NW * NW,), jnp.int32),
        pltpu.VMEM((512,), jnp.int32),
        pltpu.VMEM((WL_CAP,), jnp.int32),
        pltpu.VMEM((16,), jnp.int32),
        pltpu.VMEM((GB,), jnp.int32),
        pltpu.VMEM((GB,), jnp.int32),
        pltpu.VMEM((GB,), jnp.int32),
        pltpu.VMEM((GB,), jnp.int32),
        pltpu.VMEM((GB, D), jnp.float32),
        pltpu.VMEM((GB, D), jnp.float32),
        pltpu.VMEM((GRID_W,), jnp.float32),
        pltpu.SMEM((NW,), jnp.int32),
        pltpu.SMEM((NW,), jnp.int32),
        pltpu.SemaphoreType.DMA,
    ],
)
def scatter_max_kernel(binned_hbm, counts_hbm, feat_hbm, out_hbm,
                       cntv, chunkv, wl, bkref, gidx0, gidx1, lidb0, lidb1,
                       rows0, rows1, grid, startsm, lensm, semg):
    wid = lax.axis_index("s") * 2 + lax.axis_index("c")
    iota = lax.iota(jnp.int32, 16)
    zeros = jnp.zeros((16,), jnp.int32)
    fzeros = jnp.zeros((16,), jnp.float32)

    pltpu.sync_copy(counts_hbm, cntv)
    bkref[...] = zeros

    @pl.loop(0, NW)
    def _win(s):
        r0 = cntv[pl.ds(s * NW, 16)]
        r1 = cntv[pl.ds(s * NW + 16, 16)]
        lo0 = jnp.where(iota < wid, r0, 0)
        lo1 = jnp.where(iota + 16 < wid, r1, 0)
        o = lax.reduce_sum(lo0, (0,)) + lax.reduce_sum(lo1, (0,))
        e0 = jnp.where(iota == wid, r0, 0)
        e1 = jnp.where(iota + 16 == wid, r1, 0)
        c = lax.reduce_sum(e0, (0,)) + lax.reduce_sum(e1, (0,))
        startsm[s] = o
        lensm[s] = c

    def _prep_start(fr, bk, sub_base, gidx, lidb, rows):
        nreal_v = jnp.full((16,), jnp.minimum(bk - fr, GB), jnp.int32)
        for j in range(GB // 16):
            v = wl[pl.ds(fr + 16 * j, 16)]
            validm = (16 * j + iota) < nreal_v
            sent = (wid * GB + iota) * LIDB + (sub_base + SUB)
            pv = jnp.where(validm, v, sent)
            gidx[pl.ds(16 * j, 16)] = lax.shift_right_logical(pv, 13)
            lidb[pl.ds(16 * j, 16)] = pv & (LIDB - 1)
        pltpu.async_copy(feat_hbm.at[gidx], rows, semg)

    def _wait(gidx, rows):
        pltpu.make_async_copy(feat_hbm.at[gidx], rows, semg).wait()

    def _rmw(lidb, rows, sb_v):
        @pl.loop(0, GB, step=2)
        def _ent(e):
            e16 = pl.multiple_of(e & -16, 8)
            lv = lidb[pl.ds(e16, 16)]
            lane = e & 15
            for d in range(2):
                lid_splat = _permute(lv, jnp.full((16,), lane + d, jnp.int32))
                addr0 = (lid_splat - sb_v) * D
                ev = jnp.full((16,), e + d, jnp.int32)
                for kk in range(D // 16):
                    a = addr0 + 16 * kk + iota
                    g = plsc.load_gather(grid, [a])
                    rv = plsc.load_gather(rows, [ev, 16 * kk + iota])
                    plsc.store_scatter(grid, [a], jnp.maximum(g, rv))

    def _drain_all(bk, sub_base):
        nb = lax.shift_right_logical(bk + (GB - 1), 6)
        _prep_start(0, bk, sub_base, gidx0, lidb0, rows0)
        sb_v = jnp.full((16,), sub_base, jnp.int32)

        def _pair(bb, _):
            b1 = 2 * bb + 1
            b2 = 2 * bb + 2

            @pl.when(b1 < nb)
            def _p1():
                _prep_start(GB * b1, bk, sub_base, gidx1, lidb1, rows1)

            _wait(gidx0, rows0)
            _rmw(lidb0, rows0, sb_v)

            @pl.when(b1 < nb)
            def _p2():
                @pl.when(b2 < nb)
                def _p3():
                    _prep_start(GB * b2, bk, sub_base, gidx0, lidb0, rows0)
                _wait(gidx1, rows1)
                _rmw(lidb1, rows1, sb_v)
            return 0

        lax.fori_loop(0, lax.shift_right_logical(nb + 1, 1), _pair, 0)

    @pl.loop(0, NSUB)
    def _subpass(c):
        sub_base = c * SUB

        @pl.loop(0, SUB * D, step=256)
        def _zero(i):
            for kk in range(16):
                grid[pl.ds(i + 16 * kk, 16)] = fzeros

        @pl.loop(0, NW)
        def _per_src(s):
            o = startsm[s]
            cn = lensm[s]
            a8 = o & ~7
            end = o + cn
            nch = lax.shift_right_logical(end - a8 + 511, 9)

            def _chunk(k, _):
                base = jnp.minimum(a8 + 512 * k, PPW - 512)
                off8 = pl.multiple_of(s * PPW + base, 8)
                pltpu.sync_copy(binned_hbm.at[pl.ds(off8, 512)], chunkv)
                o_v = jnp.full((16,), o, jnp.int32)
                e_v = jnp.full((16,), end, jnp.int32)
                lo_v = jnp.full((16,), sub_base, jnp.int32)
                hi_v = jnp.full((16,), sub_base + SUB, jnp.int32)
                base_v = jnp.full((16,), base, jnp.int32)
                lane15 = jnp.full((16,), 15, jnp.int32)

                def _vec(j, bkv):
                    v = chunkv[pl.ds(j * 16, 16)]
                    posr = base_v + j * 16 + iota
                    lid13 = v & (LIDB - 1)
                    m = ((posr >= o_v) & (posr < e_v)
                         & (lid13 >= lo_v) & (lid13 < hi_v))
                    csum = plsc.cumsum(m.astype(jnp.int32))
                    pos = bkv + csum - 1
                    plsc.store_scatter(wl, [pos], v, mask=m)
                    return bkv + _permute(csum, lane15)

                bkv = lax.fori_loop(0, 32, _vec, bkref[...])
                bkref[...] = bkv
                bk = lax.reduce_max(bkv, (0,))

                @pl.when(bk >= WL_CAP - 512)
                def _flush():
                    _drain_all(bk, sub_base)
                    bkref[...] = zeros
                return 0

            lax.fori_loop(0, nch, _chunk, 0)

        bk = lax.reduce_max(bkref[...], (0,))

        @pl.when(bk > 0)
        def _final():
            _drain_all(bk, sub_base)
            bkref[...] = zeros

        out_off = pl.multiple_of((wid * RSEG + sub_base) * D, 8)
        pltpu.sync_copy(
            grid.at[pl.ds(0, SUB * D)],
            out_hbm.at[pl.ds(out_off, SUB * D)])


def segment_max_sc(feat, ids):
    binned, counts = bin_kernel(ids)
    out = scatter_max_kernel(binned, counts, feat)
    return out.reshape(NSEG, D)


def kernel(point_xyz, point_feature, point_mask, W, b):
    feat, ids = _pointnet(
        point_xyz.astype(jnp.float32),
        point_feature,
        point_mask.astype(jnp.int32),
        W,
        b,
    )
    vox = segment_max_sc(feat, ids)
    return vox.reshape(_B, _VSS[0], _VSS[1], _MLP_DIM)


# speedup vs baseline: 2.5165x; 1.2752x over previous
import functools
import math

import jax
import jax.numpy as jnp
import numpy as np
from jax import lax
from jax.experimental import pallas as pl
from jax.experimental.pallas import tpu as pltpu
from jax.experimental.pallas import tpu_sc as plsc
import dataclasses

_VOXEL_SIZE = (0.32, 0.32, 6.0)
_SPATIAL_SIZE = (-40.96, 40.96, -40.96, 40.96, 0.0, 6.0)
_B, _N, _D_FEAT, _MLP_DIM = 2, 131072, 16, 128


def _voxel_spatial_size():
    return [
        int(math.ceil((_SPATIAL_SIZE[2 * i + 1] - _SPATIAL_SIZE[2 * i]) / _VOXEL_SIZE[i] - 1e-6))
        for i in range(3)
    ]


def _voxel_origin():
    return np.array(
        [int(math.floor(_SPATIAL_SIZE[2 * i] / _VOXEL_SIZE[i])) for i in range(3)],
        dtype=np.int32,
    )


_VSS = _voxel_spatial_size()
_VOLUME = int(np.prod(_VSS))
_ORIGIN = _voxel_origin()
_SHIFT = (_VSS[1] * _VSS[2], _VSS[2], 1)

_BLK = 2048
_NP = _B * _N


def _k1_body(xb, yb, zb, mb, ids_ref, r_ref):
    i = pl.program_id(0)
    cols = (xb[0], yb[0], zb[0])
    valid = mb[0] != 0
    pid = jnp.zeros((16, 128), jnp.int32)
    for k in range(3):
        xk = cols[k]
        pvfk = jnp.floor(xk / _VOXEL_SIZE[k])
        r_ref[0, k] = xk - pvfk * _VOXEL_SIZE[k]
        pvk = pvfk.astype(jnp.int32) - int(_ORIGIN[k])
        valid = valid & (pvk >= 0) & (pvk < _VSS[k])
        pid = pid + pvk * _SHIFT[k]
    gidx = (i * _BLK + lax.broadcasted_iota(jnp.int32, (16, 128), 0) * 128
            + lax.broadcasted_iota(jnp.int32, (16, 128), 1))
    pid = (pid + (gidx // _N) * _VOLUME) * valid.astype(jnp.int32)
    ids_ref[0] = pid
    r_ref[0, 3] = 1.0 - valid.astype(jnp.float32)


def _k2_body(pf_ref, rm_ref, w16_ref, w4_ref, b_ref, feat_ref):
    acc = jax.lax.dot_general(
        pf_ref[0], w16_ref[...], (((1,), (0,)), ((), ())),
        preferred_element_type=jnp.float32,
    )
    acc = acc + jax.lax.dot_general(
        rm_ref[0], w4_ref[...], (((0,), (0,)), ((), ())),
        preferred_element_type=jnp.float32,
    )
    feat_ref[0] = jnp.maximum(acc + b_ref[0], 0.0)


def _pointnet(xyz, pf, mask_i32, W, b):
    nblk = _NP // _BLK
    xyz_t = xyz.reshape(_NP, 3).T.reshape(3, nblk, 16, 128)
    mask3 = mask_i32.reshape(nblk, 16, 128)
    ids, rm = pl.pallas_call(
        _k1_body,
        grid=(nblk,),
        in_specs=[
            pl.BlockSpec((1, 16, 128), lambda i: (i, 0, 0)) for _ in range(4)
        ],
        out_specs=[
            pl.BlockSpec((1, 16, 128), lambda i: (i, 0, 0)),
            pl.BlockSpec((1, 4, 16, 128), lambda i: (i, 0, 0, 0)),
        ],
        out_shape=[
            jax.ShapeDtypeStruct((nblk, 16, 128), jnp.int32),
            jax.ShapeDtypeStruct((nblk, 4, 16, 128), jnp.float32),
        ],
    )(xyz_t[0], xyz_t[1], xyz_t[2], mask3)
    w4 = jnp.concatenate(
        [W[_D_FEAT:], jnp.full((1, _MLP_DIM), -1e30, jnp.float32)], axis=0)
    feat = pl.pallas_call(
        _k2_body,
        grid=(nblk,),
        in_specs=[
            pl.BlockSpec((1, _BLK, _D_FEAT), lambda i: (i, 0, 0)),
            pl.BlockSpec((1, 4, _BLK), lambda i: (i, 0, 0)),
            pl.BlockSpec((_D_FEAT, _MLP_DIM), lambda i: (0, 0)),
            pl.BlockSpec((4, _MLP_DIM), lambda i: (0, 0)),
            pl.BlockSpec((1, _MLP_DIM), lambda i: (0, 0)),
        ],
        out_specs=pl.BlockSpec((1, _BLK, _MLP_DIM), lambda i: (i, 0, 0)),
        out_shape=jax.ShapeDtypeStruct((nblk, _BLK, _MLP_DIM), jnp.float32),
    )(
        pf.reshape(nblk, _BLK, _D_FEAT),
        rm.reshape(nblk, 4, _BLK),
        W[:_D_FEAT],
        w4,
        b.reshape(1, _MLP_DIM),
    )
    return feat.reshape(_NP, _MLP_DIM), ids.reshape(_NP)


NW = 32
NP = 262144
PPW = NP // NW
NSEG = 131072
RSEG = NSEG // NW
SUB = 512
NSUB = RSEG // SUB
D = 128
NBK = 256
LIDB = 1024

_MESH = plsc.VectorSubcoreMesh(core_axis_name="c", subcore_axis_name="s")
_CP = pltpu.CompilerParams()
if "needs_layout_passes" in pltpu.CompilerParams.__dataclass_fields__:
    _CP = dataclasses.replace(_CP, needs_layout_passes=False)


def _permute(x, idx):
    dnums = lax.GatherDimensionNumbers(
        offset_dims=(), collapsed_slice_dims=(0,), start_index_map=(0,))
    return lax.gather(x, idx[:, None], dnums, (1,),
                      mode=lax.GatherScatterMode.PROMISE_IN_BOUNDS)


@functools.partial(
    pl.kernel,
    out_type=[
        jax.ShapeDtypeStruct((NP,), jnp.int32),
        jax.ShapeDtypeStruct((NW * NBK,), jnp.int32),
    ],
    mesh=_MESH,
    compiler_params=_CP,
    scratch_types=[
        pltpu.VMEM((PPW,), jnp.int32),
        pltpu.VMEM((PPW,), jnp.int32),
        pltpu.VMEM((NBK,), jnp.int32),
        pltpu.VMEM((NBK,), jnp.int32),
        pltpu.SemaphoreType.DMA,
    ],
)
def bin_kernel(ids_hbm, binned_hbm, counts_hbm, idsv, binv, cnt, off, sem):
    wid = lax.axis_index("s") * 2 + lax.axis_index("c")
    iota = lax.iota(jnp.int32, 16)
    zeros = jnp.zeros((16,), jnp.int32)
    ones = jnp.ones((16,), jnp.int32)

    pltpu.sync_copy(ids_hbm.at[pl.ds(wid * PPW, PPW)], idsv)

    @pl.loop(0, NBK, step=16)
    def _z(i):
        cnt[pl.ds(i, 16)] = zeros

    @pl.loop(0, PPW, step=16)
    def _hist(i):
        v = idsv[pl.ds(i, 16)]
        r = lax.shift_right_logical(v, 9)
        plsc.addupdate_scatter(cnt, [r], ones)

    def _pfx(j, carry):
        cj = cnt[pl.ds(pl.multiple_of(j * 16, 8), 16)]
        sj = plsc.cumsum(cj)
        off[pl.ds(pl.multiple_of(j * 16, 8), 16)] = carry + sj - cj
        return carry + _permute(sj, jnp.full((16,), 15, jnp.int32))

    lax.fori_loop(0, NBK // 16, _pfx, zeros)

    @pl.loop(0, PPW, step=16)
    def _place(i):
        v = idsv[pl.ds(i, 16)]
        r = lax.shift_right_logical(v, 9)
        pidx = wid * PPW + i + iota
        packed = pidx * LIDB + (v & 511)
        skey, spacked = plsc.sort_key_val(r, packed)
        prev = _permute(skey, jnp.maximum(iota - 1, 0))
        boundary = (iota == 0) | (skey != prev)
        starts = jnp.where(boundary, iota, 0)
        rank = iota - plsc.cummax(starts)
        base = plsc.load_gather(off, [skey])
        plsc.store_scatter(binv, [base + rank], spacked)
        plsc.addupdate_scatter(off, [skey], ones)

    pltpu.sync_copy(binv, binned_hbm.at[pl.ds(wid * PPW, PPW)])
    pltpu.sync_copy(cnt, counts_hbm.at[pl.ds(wid * NBK, NBK)])


WL_CAP = 2560
GB = 64
GRID_W = (SUB + 1) * D


@functools.partial(
    pl.kernel,
    out_type=jax.ShapeDtypeStruct((NSEG * D,), jnp.float32),
    mesh=_MESH,
    compiler_params=_CP,
    scratch_types=[
        pltpu.VMEM((NW * NBK,), jnp.int32),
        pltpu.VMEM((512,), jnp.int32),
        pltpu.VMEM((WL_CAP,), jnp.int32),
        pltpu.VMEM((16,), jnp.int32),
        pltpu.VMEM((GB,), jnp.int32),
        pltpu.VMEM((GB,), jnp.int32),
        pltpu.VMEM((GB,), jnp.int32),
        pltpu.VMEM((GB,), jnp.int32),
        pltpu.VMEM((GB, D), jnp.float32),
        pltpu.VMEM((GB, D), jnp.float32),
        pltpu.VMEM((GRID_W,), jnp.float32),
        pltpu.SMEM((NW * 8,), jnp.int32),
        pltpu.SMEM((NW * 8,), jnp.int32),
        pltpu.SemaphoreType.DMA,
    ],
)
def scatter_max_kernel(binned_hbm, counts_hbm, feat_hbm, out_hbm,
                       cntv, chunkv, wl, bkref, gidx0, gidx1, lidb0, lidb1,
                       rows0, rows1, grid, startsm, lensm, semg):
    wid = lax.axis_index("s") * 2 + lax.axis_index("c")
    iota = lax.iota(jnp.int32, 16)
    zeros = jnp.zeros((16,), jnp.int32)
    fzeros = jnp.zeros((16,), jnp.float32)
    b0 = wid * 8

    pltpu.sync_copy(counts_hbm, cntv)
    bkref[...] = zeros

    @pl.loop(0, NW)
    def _win(s):
        def _base(j, acc):
            cj = cntv[pl.ds(pl.multiple_of(s * NBK + j * 16, 8), 16)]
            m = (j * 16 + iota) < jnp.full((16,), b0, jnp.int32)
            return acc + lax.reduce_sum(jnp.where(m, cj, 0), (0,))

        base = lax.fori_loop(0, NBK // 16, _base, jnp.int32(0))
        v8 = cntv[pl.ds(pl.multiple_of(s * NBK + (b0 & ~15), 8), 16)]
        l8 = b0 & 15

        def _w1(c, acc):
            lc = lax.reduce_sum(
                jnp.where(iota == l8 + c, v8, 0), (0,))
            startsm[s * 8 + c] = acc
            lensm[s * 8 + c] = lc
            return acc + lc

        lax.fori_loop(0, 8, _w1, base)

    def _prep_start(fr, bk, gidx, lidb, rows):
        nreal_v = jnp.full((16,), jnp.minimum(bk - fr, GB), jnp.int32)
        for j in range(GB // 16):
            v = wl[pl.ds(fr + 16 * j, 16)]
            validm = (16 * j + iota) < nreal_v
            sent = (wid * GB + iota) * LIDB + SUB
            pv = jnp.where(validm, v, sent)
            gidx[pl.ds(16 * j, 16)] = lax.shift_right_logical(pv, 10)
            lidb[pl.ds(16 * j, 16)] = pv & (LIDB - 1)
        pltpu.async_copy(feat_hbm.at[gidx], rows, semg)

    def _wait(gidx, rows):
        pltpu.make_async_copy(feat_hbm.at[gidx], rows, semg).wait()

    def _rmw(lidb, rows):
        @pl.loop(0, GB, step=2)
        def _ent(e):
            e16 = pl.multiple_of(e & -16, 8)
            lv = lidb[pl.ds(e16, 16)]
            lane = e & 15
            for d in range(2):
                lid_splat = _permute(lv, jnp.full((16,), lane + d, jnp.int32))
                addr0 = lid_splat * D
                ev = jnp.full((16,), e + d, jnp.int32)
                for kk in range(D // 16):
                    a = addr0 + 16 * kk + iota
                    g = plsc.load_gather(grid, [a])
                    rv = plsc.load_gather(rows, [ev, 16 * kk + iota])
                    plsc.store_scatter(grid, [a], jnp.maximum(g, rv))

    def _drain_all(bk):
        nb = lax.shift_right_logical(bk + (GB - 1), 6)
        _prep_start(0, bk, gidx0, lidb0, rows0)

        def _pair(bb, _):
            b1 = 2 * bb + 1
            b2 = 2 * bb + 2

            @pl.when(b1 < nb)
            def _p1():
                _prep_start(GB * b1, bk, gidx1, lidb1, rows1)

            _wait(gidx0, rows0)
            _rmw(lidb0, rows0)

            @pl.when(b1 < nb)
            def _p2():
                @pl.when(b2 < nb)
                def _p3():
                    _prep_start(GB * b2, bk, gidx0, lidb0, rows0)
                _wait(gidx1, rows1)
                _rmw(lidb1, rows1)
            return 0

        lax.fori_loop(0, lax.shift_right_logical(nb + 1, 1), _pair, 0)

    @pl.loop(0, 8)
    def _bucket(c):
        @pl.loop(0, SUB * D, step=256)
        def _zero(i):
            for kk in range(16):
                grid[pl.ds(i + 16 * kk, 16)] = fzeros

        @pl.loop(0, NW)
        def _per_src(s):
            o = startsm[s * 8 + c]
            cn = lensm[s * 8 + c]
            a8 = o & ~7
            end = o + cn
            nch = lax.shift_right_logical(end - a8 + 511, 9)

            def _chunk(k, _):
                base = jnp.minimum(a8 + 512 * k, PPW - 512)
                off8 = pl.multiple_of(s * PPW + base, 8)
                pltpu.sync_copy(binned_hbm.at[pl.ds(off8, 512)], chunkv)
                o_v = jnp.full((16,), o, jnp.int32)
                e_v = jnp.full((16,), end, jnp.int32)
                base_v = jnp.full((16,), base, jnp.int32)
                lane15 = jnp.full((16,), 15, jnp.int32)
                nvec = lax.shift_right_logical(
                    jnp.minimum(end - base, 512) + 15, 4)

                def _vec(j, bkv):
                    v = chunkv[pl.ds(j * 16, 16)]
                    posr = base_v + j * 16 + iota
                    m = (posr >= o_v) & (posr < e_v)
                    csum = plsc.cumsum(m.astype(jnp.int32))
                    pos = bkv + csum - 1
                    plsc.store_scatter(wl, [pos], v, mask=m)
                    return bkv + _permute(csum, lane15)

                bkv = lax.fori_loop(0, nvec, _vec, bkref[...])
                bkref[...] = bkv
                bk = lax.reduce_max(bkv, (0,))

                @pl.when(bk >= WL_CAP - 512)
                def _flush():
                    _drain_all(bk)
                    bkref[...] = zeros
                return 0

            lax.fori_loop(0, nch, _chunk, 0)

        bk = lax.reduce_max(bkref[...], (0,))

        @pl.when(bk > 0)
        def _final():
            _drain_all(bk)
            bkref[...] = zeros

        out_off = pl.multiple_of((b0 + c) * SUB * D, 8)
        pltpu.sync_copy(
            grid.at[pl.ds(0, SUB * D)],
            out_hbm.at[pl.ds(out_off, SUB * D)])


def segment_max_sc(feat, ids):
    binned, counts = bin_kernel(ids)
    out = scatter_max_kernel(binned, counts, feat)
    return out.reshape(NSEG, D)


def kernel(point_xyz, point_feature, point_mask, W, b):
    feat, ids = _pointnet(
        point_xyz.astype(jnp.float32),
        point_feature,
        point_mask.astype(jnp.int32),
        W,
        b,
    )
    vox = segment_max_sc(feat, ids)
    return vox.reshape(_B, _VSS[0], _VSS[1], _MLP_DIM)


# speedup vs baseline: 2.9088x; 1.1559x over previous
import functools
import math

import jax
import jax.numpy as jnp
import numpy as np
from jax import lax
from jax.experimental import pallas as pl
from jax.experimental.pallas import tpu as pltpu
from jax.experimental.pallas import tpu_sc as plsc
import dataclasses

_VOXEL_SIZE = (0.32, 0.32, 6.0)
_SPATIAL_SIZE = (-40.96, 40.96, -40.96, 40.96, 0.0, 6.0)
_B, _N, _D_FEAT, _MLP_DIM = 2, 131072, 16, 128


def _voxel_spatial_size():
    return [
        int(math.ceil((_SPATIAL_SIZE[2 * i + 1] - _SPATIAL_SIZE[2 * i]) / _VOXEL_SIZE[i] - 1e-6))
        for i in range(3)
    ]


def _voxel_origin():
    return np.array(
        [int(math.floor(_SPATIAL_SIZE[2 * i] / _VOXEL_SIZE[i])) for i in range(3)],
        dtype=np.int32,
    )


_VSS = _voxel_spatial_size()
_VOLUME = int(np.prod(_VSS))
_ORIGIN = _voxel_origin()
_SHIFT = (_VSS[1] * _VSS[2], _VSS[2], 1)

_BLK = 8192
_SL = _BLK // 128
_NP = _B * _N


def _k1_body(xb, yb, zb, mb, ids_ref, r_ref):
    i = pl.program_id(0)
    cols = (xb[0], yb[0], zb[0])
    valid = mb[0] != 0
    pid = jnp.zeros((_SL, 128), jnp.int32)
    for k in range(3):
        xk = cols[k]
        pvfk = jnp.floor(xk / _VOXEL_SIZE[k])
        r_ref[0, k] = xk - pvfk * _VOXEL_SIZE[k]
        pvk = pvfk.astype(jnp.int32) - int(_ORIGIN[k])
        valid = valid & (pvk >= 0) & (pvk < _VSS[k])
        pid = pid + pvk * _SHIFT[k]
    gidx = (i * _BLK + lax.broadcasted_iota(jnp.int32, (_SL, 128), 0) * 128
            + lax.broadcasted_iota(jnp.int32, (_SL, 128), 1))
    pid = (pid + (gidx // _N) * _VOLUME) * valid.astype(jnp.int32)
    ids_ref[0] = pid
    r_ref[0, 3] = 1.0 - valid.astype(jnp.float32)


def _k2_body(pf_ref, rm_ref, w16_ref, w4_ref, b_ref, feat_ref):
    acc = jax.lax.dot_general(
        pf_ref[0], w16_ref[...], (((1,), (0,)), ((), ())),
        preferred_element_type=jnp.float32,
    )
    acc = acc + jax.lax.dot_general(
        rm_ref[0], w4_ref[...], (((0,), (0,)), ((), ())),
        preferred_element_type=jnp.float32,
    )
    feat_ref[0] = jnp.maximum(acc + b_ref[0], 0.0)


def _pointnet(xyz, pf, mask_i32, W, b):
    nblk = _NP // _BLK
    xyz_t = xyz.reshape(_NP, 3).T.reshape(3, nblk, _SL, 128)
    mask3 = mask_i32.reshape(nblk, _SL, 128)
    ids, rm = pl.pallas_call(
        _k1_body,
        grid=(nblk,),
        in_specs=[
            pl.BlockSpec((1, _SL, 128), lambda i: (i, 0, 0)) for _ in range(4)
        ],
        out_specs=[
            pl.BlockSpec((1, _SL, 128), lambda i: (i, 0, 0)),
            pl.BlockSpec((1, 4, _SL, 128), lambda i: (i, 0, 0, 0)),
        ],
        out_shape=[
            jax.ShapeDtypeStruct((nblk, _SL, 128), jnp.int32),
            jax.ShapeDtypeStruct((nblk, 4, _SL, 128), jnp.float32),
        ],
    )(xyz_t[0], xyz_t[1], xyz_t[2], mask3)
    w4 = jnp.concatenate(
        [W[_D_FEAT:], jnp.full((1, _MLP_DIM), -1e30, jnp.float32)], axis=0)
    feat = pl.pallas_call(
        _k2_body,
        grid=(nblk,),
        in_specs=[
            pl.BlockSpec((1, _BLK, _D_FEAT), lambda i: (i, 0, 0)),
            pl.BlockSpec((1, 4, _BLK), lambda i: (i, 0, 0)),
            pl.BlockSpec((_D_FEAT, _MLP_DIM), lambda i: (0, 0)),
            pl.BlockSpec((4, _MLP_DIM), lambda i: (0, 0)),
            pl.BlockSpec((1, _MLP_DIM), lambda i: (0, 0)),
        ],
        out_specs=pl.BlockSpec((1, _BLK, _MLP_DIM), lambda i: (i, 0, 0)),
        out_shape=jax.ShapeDtypeStruct((nblk, _BLK, _MLP_DIM), jnp.float32),
    )(
        pf.reshape(nblk, _BLK, _D_FEAT),
        rm.reshape(nblk, 4, _BLK),
        W[:_D_FEAT],
        w4,
        b.reshape(1, _MLP_DIM),
    )
    return feat.reshape(_NP, _MLP_DIM), ids.reshape(_NP)


NW = 32
NP = 262144
PPW = NP // NW
NSEG = 131072
RSEG = NSEG // NW
SUB = 512
NSUB = RSEG // SUB
D = 128
NBK = 256
LIDB = 1024

_MESH = plsc.VectorSubcoreMesh(core_axis_name="c", subcore_axis_name="s")
_CP = pltpu.CompilerParams()
if "needs_layout_passes" in pltpu.CompilerParams.__dataclass_fields__:
    _CP = dataclasses.replace(_CP, needs_layout_passes=False)


def _permute(x, idx):
    dnums = lax.GatherDimensionNumbers(
        offset_dims=(), collapsed_slice_dims=(0,), start_index_map=(0,))
    return lax.gather(x, idx[:, None], dnums, (1,),
                      mode=lax.GatherScatterMode.PROMISE_IN_BOUNDS)


@functools.partial(
    pl.kernel,
    out_type=[
        jax.ShapeDtypeStruct((NP,), jnp.int32),
        jax.ShapeDtypeStruct((NW * NBK,), jnp.int32),
    ],
    mesh=_MESH,
    compiler_params=_CP,
    scratch_types=[
        pltpu.VMEM((PPW,), jnp.int32),
        pltpu.VMEM((PPW,), jnp.int32),
        pltpu.VMEM((NBK,), jnp.int32),
        pltpu.VMEM((NBK,), jnp.int32),
        pltpu.SemaphoreType.DMA,
    ],
)
def bin_kernel(ids_hbm, binned_hbm, counts_hbm, idsv, binv, cnt, off, sem):
    wid = lax.axis_index("s") * 2 + lax.axis_index("c")
    iota = lax.iota(jnp.int32, 16)
    zeros = jnp.zeros((16,), jnp.int32)
    ones = jnp.ones((16,), jnp.int32)

    pltpu.sync_copy(ids_hbm.at[pl.ds(wid * PPW, PPW)], idsv)

    @pl.loop(0, NBK, step=16)
    def _z(i):
        cnt[pl.ds(i, 16)] = zeros

    @pl.loop(0, PPW, step=16)
    def _hist(i):
        v = idsv[pl.ds(i, 16)]
        r = lax.shift_right_logical(v, 9)
        plsc.addupdate_scatter(cnt, [r], ones)

    def _pfx(j, carry):
        cj = cnt[pl.ds(pl.multiple_of(j * 16, 8), 16)]
        sj = plsc.cumsum(cj)
        off[pl.ds(pl.multiple_of(j * 16, 8), 16)] = carry + sj - cj
        return carry + _permute(sj, jnp.full((16,), 15, jnp.int32))

    lax.fori_loop(0, NBK // 16, _pfx, zeros)

    @pl.loop(0, PPW, step=16)
    def _place(i):
        v = idsv[pl.ds(i, 16)]
        r = lax.shift_right_logical(v, 9)
        pidx = wid * PPW + i + iota
        packed = pidx * LIDB + (v & 511)
        skey, spacked = plsc.sort_key_val(r, packed)
        prev = _permute(skey, jnp.maximum(iota - 1, 0))
        boundary = (iota == 0) | (skey != prev)
        starts = jnp.where(boundary, iota, 0)
        rank = iota - plsc.cummax(starts)
        base = plsc.load_gather(off, [skey])
        plsc.store_scatter(binv, [base + rank], spacked)
        plsc.addupdate_scatter(off, [skey], ones)

    pltpu.sync_copy(binv, binned_hbm.at[pl.ds(wid * PPW, PPW)])
    pltpu.sync_copy(cnt, counts_hbm.at[pl.ds(wid * NBK, NBK)])


WL_CAP = 2560
GB = 64
GRID_W = (SUB + 1) * D


@functools.partial(
    pl.kernel,
    out_type=jax.ShapeDtypeStruct((NSEG * D,), jnp.float32),
    mesh=_MESH,
    compiler_params=_CP,
    scratch_types=[
        pltpu.VMEM((NW * NBK,), jnp.int32),
        pltpu.VMEM((512,), jnp.int32),
        pltpu.VMEM((WL_CAP,), jnp.int32),
        pltpu.VMEM((16,), jnp.int32),
        pltpu.VMEM((GB,), jnp.int32),
        pltpu.VMEM((GB,), jnp.int32),
        pltpu.VMEM((GB,), jnp.int32),
        pltpu.VMEM((GB,), jnp.int32),
        pltpu.VMEM((GB, D), jnp.float32),
        pltpu.VMEM((GB, D), jnp.float32),
        pltpu.VMEM((GRID_W,), jnp.float32),
        pltpu.SMEM((NW * 8,), jnp.int32),
        pltpu.SMEM((NW * 8,), jnp.int32),
        pltpu.SemaphoreType.DMA,
    ],
)
def scatter_max_kernel(binned_hbm, counts_hbm, feat_hbm, out_hbm,
                       cntv, chunkv, wl, bkref, gidx0, gidx1, lidb0, lidb1,
                       rows0, rows1, grid, startsm, lensm, semg):
    wid = lax.axis_index("s") * 2 + lax.axis_index("c")
    iota = lax.iota(jnp.int32, 16)
    zeros = jnp.zeros((16,), jnp.int32)
    fzeros = jnp.zeros((16,), jnp.float32)
    b0 = wid * 8

    pltpu.sync_copy(counts_hbm, cntv)
    bkref[...] = zeros

    @pl.loop(0, NW)
    def _win(s):
        def _base(j, acc):
            cj = cntv[pl.ds(pl.multiple_of(s * NBK + j * 16, 8), 16)]
            m = (j * 16 + iota) < jnp.full((16,), b0, jnp.int32)
            return acc + lax.reduce_sum(jnp.where(m, cj, 0), (0,))

        base = lax.fori_loop(0, NBK // 16, _base, jnp.int32(0))
        v8 = cntv[pl.ds(pl.multiple_of(s * NBK + (b0 & ~15), 8), 16)]
        l8 = b0 & 15

        def _w1(c, acc):
            lc = lax.reduce_sum(
                jnp.where(iota == l8 + c, v8, 0), (0,))
            startsm[s * 8 + c] = acc
            lensm[s * 8 + c] = lc
            return acc + lc

        lax.fori_loop(0, 8, _w1, base)

    def _prep_start(fr, bk, gidx, lidb, rows):
        nreal_v = jnp.full((16,), jnp.minimum(bk - fr, GB), jnp.int32)
        for j in range(GB // 16):
            v = wl[pl.ds(fr + 16 * j, 16)]
            validm = (16 * j + iota) < nreal_v
            sent = (wid * GB + iota) * LIDB + SUB
            pv = jnp.where(validm, v, sent)
            gidx[pl.ds(16 * j, 16)] = lax.shift_right_logical(pv, 10)
            lidb[pl.ds(16 * j, 16)] = pv & (LIDB - 1)
        pltpu.async_copy(feat_hbm.at[gidx], rows, semg)

    def _wait(gidx, rows):
        pltpu.make_async_copy(feat_hbm.at[gidx], rows, semg).wait()

    def _rmw(lidb, rows):
        @pl.loop(0, GB, step=2)
        def _ent(e):
            e16 = pl.multiple_of(e & -16, 8)
            lv = lidb[pl.ds(e16, 16)]
            lane = e & 15
            for d in range(2):
                lid_splat = _permute(lv, jnp.full((16,), lane + d, jnp.int32))
                addr0 = lid_splat * D
                ev = jnp.full((16,), e + d, jnp.int32)
                for kk in range(D // 16):
                    a = addr0 + 16 * kk + iota
                    g = plsc.load_gather(grid, [a])
                    rv = plsc.load_gather(rows, [ev, 16 * kk + iota])
                    plsc.store_scatter(grid, [a], jnp.maximum(g, rv))

    def _drain_all(bk):
        nb = lax.shift_right_logical(bk + (GB - 1), 6)
        _prep_start(0, bk, gidx0, lidb0, rows0)

        def _pair(bb, _):
            b1 = 2 * bb + 1
            b2 = 2 * bb + 2

            @pl.when(b1 < nb)
            def _p1():
                _prep_start(GB * b1, bk, gidx1, lidb1, rows1)

            _wait(gidx0, rows0)
            _rmw(lidb0, rows0)

            @pl.when(b1 < nb)
            def _p2():
                @pl.when(b2 < nb)
                def _p3():
                    _prep_start(GB * b2, bk, gidx0, lidb0, rows0)
                _wait(gidx1, rows1)
                _rmw(lidb1, rows1)
            return 0

        lax.fori_loop(0, lax.shift_right_logical(nb + 1, 1), _pair, 0)

    @pl.loop(0, 8)
    def _bucket(c):
        @pl.loop(0, SUB * D, step=256)
        def _zero(i):
            for kk in range(16):
                grid[pl.ds(i + 16 * kk, 16)] = fzeros

        @pl.loop(0, NW)
        def _per_src(s):
            o = startsm[s * 8 + c]
            cn = lensm[s * 8 + c]
            a8 = o & ~7
            end = o + cn
            nch = lax.shift_right_logical(end - a8 + 511, 9)

            def _chunk(k, _):
                base = jnp.minimum(a8 + 512 * k, PPW - 512)
                off8 = pl.multiple_of(s * PPW + base, 8)
                pltpu.sync_copy(binned_hbm.at[pl.ds(off8, 512)], chunkv)
                o_v = jnp.full((16,), o, jnp.int32)
                e_v = jnp.full((16,), end, jnp.int32)
                base_v = jnp.full((16,), base, jnp.int32)
                lane15 = jnp.full((16,), 15, jnp.int32)
                nvec = lax.shift_right_logical(
                    jnp.minimum(end - base, 512) + 15, 4)

                def _vec(j, bkv):
                    v = chunkv[pl.ds(j * 16, 16)]
                    posr = base_v + j * 16 + iota
                    m = (posr >= o_v) & (posr < e_v)
                    csum = plsc.cumsum(m.astype(jnp.int32))
                    pos = bkv + csum - 1
                    plsc.store_scatter(wl, [pos], v, mask=m)
                    return bkv + _permute(csum, lane15)

                bkv = lax.fori_loop(0, nvec, _vec, bkref[...])
                bkref[...] = bkv
                bk = lax.reduce_max(bkv, (0,))

                @pl.when(bk >= WL_CAP - 512)
                def _flush():
                    _drain_all(bk)
                    bkref[...] = zeros
                return 0

            lax.fori_loop(0, nch, _chunk, 0)

        bk = lax.reduce_max(bkref[...], (0,))

        @pl.when(bk > 0)
        def _final():
            _drain_all(bk)
            bkref[...] = zeros

        out_off = pl.multiple_of((b0 + c) * SUB * D, 8)
        pltpu.sync_copy(
            grid.at[pl.ds(0, SUB * D)],
            out_hbm.at[pl.ds(out_off, SUB * D)])


def segment_max_sc(feat, ids):
    binned, counts = bin_kernel(ids)
    out = scatter_max_kernel(binned, counts, feat)
    return out.reshape(NSEG, D)


def kernel(point_xyz, point_feature, point_mask, W, b):
    feat, ids = _pointnet(
        point_xyz.astype(jnp.float32),
        point_feature,
        point_mask.astype(jnp.int32),
        W,
        b,
    )
    vox = segment_max_sc(feat, ids)
    return vox.reshape(_B, _VSS[0], _VSS[1], _MLP_DIM)
